# baseline (speedup 1.0000x reference)
import functools

import jax
import jax.numpy as jnp
from jax import lax
from jax.experimental import pallas as pl
from jax.experimental.pallas import tpu as pltpu
from jax.experimental.pallas import tpu_sc as plsc

VOCAB = 100000
MAXLEN = 200
EMBED = 32
BATCH = 4096

NC = 2
NS = 16
NW = NC * NS
STILE = MAXLEN // 8
CB = BATCH // 128
SPC = 4
NCHUNK = MAXLEN // SPC
EG = EMBED // 4
NB = 4

_mesh = plsc.VectorSubcoreMesh(core_axis_name="c", subcore_axis_name="s")


@functools.partial(
    pl.kernel,
    mesh=_mesh,
    compiler_params=pltpu.CompilerParams(
        use_tc_tiling_on_sc=False, needs_layout_passes=False
    ),
    out_type=jax.ShapeDtypeStruct((MAXLEN, 4, CB, 1024), jnp.float32),
    scratch_types=[
        pltpu.VMEM((NB, SPC, 128), jnp.int32),
        pltpu.VMEM((NB, SPC * 128, EMBED), jnp.float32),
        pltpu.VMEM((2, SPC * EMBED * 128), jnp.float32),
        pltpu.VMEM((MAXLEN * EMBED,), jnp.float32),
        pltpu.SemaphoreType.DMA,
        pltpu.SemaphoreType.DMA,
        pltpu.SemaphoreType.DMA,
        pltpu.SemaphoreType.DMA,
        pltpu.SemaphoreType.DMA,
        pltpu.SemaphoreType.DMA,
        pltpu.SemaphoreType.DMA,
        pltpu.SemaphoreType.DMA,
        pltpu.SemaphoreType.DMA,
        pltpu.SemaphoreType.DMA,
    ],
)
def _emb(
    xv_hbm, tok_hbm, pos_hbm, out_hbm,
    idx_v, gbuf, tbuf, pos_v,
    isem0, isem1, isem2, isem3,
    gsem0, gsem1, gsem2, gsem3,
    osem0, osem1,
):
    isem = (isem0, isem1, isem2, isem3)
    gsem = (gsem0, gsem1, gsem2, gsem3)
    osem = (osem0, osem1)
    c = lax.axis_index("s") * NC + lax.axis_index("c")
    pltpu.sync_copy(pos_hbm, pos_v)

    def issue_idx(m, q):
        pltpu.async_copy(
            xv_hbm.at[m // 2, c, pl.ds(lax.rem(m, 2) * SPC, SPC)],
            idx_v.at[q],
            isem[q],
        )

    def wait_idx(q):
        pltpu.make_async_copy(
            xv_hbm.at[0, 0, pl.ds(0, SPC)], idx_v.at[q], isem[q]
        ).wait()

    def issue_gathers(q):
        for sr in range(SPC):
            pltpu.async_copy(
                tok_hbm.at[idx_v.at[q, sr]],
                gbuf.at[q, pl.ds(sr * 128, 128)],
                gsem[q],
            )

    def wait_gathers(q):
        pltpu.make_async_copy(
            tok_hbm.at[pl.ds(0, SPC * 128)], gbuf.at[q], gsem[q]
        ).wait()

    def transpose_add(q, t, s0):
        qconst = jnp.full((16,), q, jnp.int32)
        tconst = jnp.full((16,), t, jnp.int32)

        @plsc.parallel_loop(0, EMBED, unroll=1)
        def d_body(d):
            iota = lax.iota(jnp.int32, 16)
            e_par = [lax.bitwise_and(iota + (par * 16) + d, EMBED - 1)
                     for par in range(2)]
            se_par = [ev << 7 for ev in e_par]
            pos_par = [
                [plsc.load_gather(pos_v, [ev + ((s0 + sr) * EMBED)])
                 for ev in e_par]
                for sr in range(SPC)
            ]
            for lb in range(8):
                par = lb % 2
                b_vec = iota + (lb * 16)
                e_vec = e_par[par]
                sidx = se_par[par] + b_vec
                for sr in range(SPC):
                    row_vec = b_vec + (sr * 128)
                    vals = plsc.load_gather(gbuf, [qconst, row_vec, e_vec])
                    plsc.store_scatter(
                        tbuf,
                        [tconst, sidx + (sr * EMBED * 128)],
                        vals + pos_par[sr][par],
                    )

    def issue_out(t, s0):
        for sr in range(SPC):
            for g in range(4):
                pltpu.async_copy(
                    tbuf.at[t, pl.ds(sr * EMBED * 128 + g * 1024, 1024)],
                    out_hbm.at[s0 + sr, g, c],
                    osem[t],
                )

    def wait_out(t):
        for sr in range(SPC):
            for g in range(4):
                pltpu.make_async_copy(
                    tbuf.at[t, pl.ds(g * 1024, 1024)],
                    out_hbm.at[sr, g, c],
                    osem[t],
                ).wait()

    issue_idx(0, 0)
    issue_idx(1, 1)
    issue_idx(2, 2)
    issue_idx(3, 3)
    wait_idx(0)
    issue_gathers(0)
    wait_idx(1)
    issue_gathers(1)

    def step(k, q, t):
        wait_idx((q + 2) % NB)
        issue_gathers((q + 2) % NB)
        @pl.when(k >= 2)
        def _():
            wait_out(t)
        wait_gathers(q)
        @pl.when(k < NCHUNK - 4)
        def _():
            issue_idx(k + 4, q)
        transpose_add(q, t, k * SPC)
        issue_out(t, k * SPC)

    def loop_body(i, carry):
        for qq in range(NB):
            step(i * NB + qq, qq, qq % 2)
        return carry

    lax.fori_loop(0, (NCHUNK - 2) // NB, loop_body, 0)
    for k in (NCHUNK - 2, NCHUNK - 1):
        q = k % NB
        t = k % 2
        wait_out(t)
        wait_gathers(q)
        transpose_add(q, t, k * SPC)
        issue_out(t, k * SPC)
    wait_out(0)
    wait_out(1)


def kernel(x, token_table, pos_table):
    xv = (
        x.astype(jnp.int32)
        .T.reshape(STILE, 8, CB, 128)
        .transpose(0, 2, 1, 3)
    )
    out4 = _emb(xv, token_table, pos_table.reshape(-1))
    return (
        out4.reshape(MAXLEN, 4, CB, 8, 128)
        .transpose(2, 4, 0, 1, 3)
        .reshape(BATCH, MAXLEN, EMBED)
    )
